# Initial kernel scaffold; baseline (speedup 1.0000x reference)
import jax
import jax.numpy as jnp
from jax.experimental import pallas as pl


def kernel(x, edge_index, batch, batch_size, Wr, Wn, b, lin1_w, lin1_b, lin2_w, lin2_b, linc_w, linc_b):
    raise NotImplementedError("write your pallas kernel here")


# speedup vs baseline: 10.4668x; 10.4668x over previous
import functools

import jax
import jax.numpy as jnp
from jax import lax
from jax.experimental import pallas as pl
from jax.experimental.pallas import tpu as pltpu
from jax.experimental.pallas import tpu_sc as plsc

N, E, D, H, C, L, B = 10000, 320000, 128, 128, 10, 3, 16

NC, NS = 2, 16
NW = NC * NS
EW = E // NW
K = 80
CH = EW // K
NP = 10240
ROWS = NP // NS

G = 5
SLABS, CHS = 5, 25

BL = 1000

_mesh = plsc.VectorSubcoreMesh(
    core_axis_name="c", subcore_axis_name="s", num_cores=NC, num_subcores=NS)


def _sc_deg_body(dstr, z128, ones_hbm,
                 deg_out,
                 dst_v, ones_v, deg_s, ssem):
    c = lax.axis_index("c")
    s = lax.axis_index("s")
    wid = s * NC + c
    pltpu.sync_copy(z128.at[pl.ds(s * ROWS, ROWS)], deg_s.at[pl.ds(s * ROWS, ROWS)])
    pltpu.sync_copy(dstr.at[wid], dst_v)
    pltpu.sync_copy(ones_hbm, ones_v)
    plsc.subcore_barrier()

    def dgroup(base, drain):
        for k in range(G):
            if drain:
                pltpu.make_async_copy(ones_v, deg_s.at[dst_v.at[base + k - G]],
                                      ssem).wait()
            pltpu.async_copy(ones_v, deg_s.at[dst_v.at[base + k]], ssem,
                             add=True)

    dgroup(0, False)

    def body(jj, carry):
        dgroup(jj * G, True)
        return carry

    lax.fori_loop(1, CH // G, body, 0)
    for k in range(G):
        pltpu.make_async_copy(ones_v, deg_s.at[dst_v.at[CH - G + k]],
                              ssem).wait()
    plsc.subcore_barrier()
    base = c * NP + s * ROWS
    pltpu.sync_copy(deg_s.at[pl.ds(s * ROWS, ROWS)], deg_out.at[pl.ds(base, ROWS)])


_sc_deg = functools.partial(
    pl.kernel,
    out_type=jax.ShapeDtypeStruct((NC * NP, D), jnp.float32),
    mesh=_mesh,
    scratch_types=[
        pltpu.VMEM((CH, K), jnp.int32),
        pltpu.VMEM((K, D), jnp.float32),
        pltpu.VMEM_SHARED((NP, D), jnp.float32),
        pltpu.SemaphoreType.DMA,
    ],
)(_sc_deg_body)


def _sc_agg_body(h_hbm, srcr, dstr, z128,
                 agg_out,
                 src_v, dst_v, r0, r1, r2, r3, agg_s,
                 g0, g1, g2, g3, s0_, s1_, s2_, s3_):
    rows = (r0, r1, r2, r3)
    gsems = (g0, g1, g2, g3)
    c = lax.axis_index("c")
    s = lax.axis_index("s")
    wid = s * NC + c
    pltpu.sync_copy(z128.at[pl.ds(s * ROWS, ROWS)], agg_s.at[pl.ds(s * ROWS, ROWS)])
    plsc.subcore_barrier()

    ssems = (s0_, s1_, s2_, s3_)

    def group(base, drain):
        cps = []
        for k in range(4):
            if drain:
                pltpu.make_async_copy(rows[k],
                                      agg_s.at[dst_v.at[base + k - 4]],
                                      ssems[k]).wait()
            cps.append(pltpu.async_copy(h_hbm.at[src_v.at[base + k]], rows[k],
                                        gsems[k]))
        for k in range(4):
            cps[k].wait()
            pltpu.async_copy(rows[k], agg_s.at[dst_v.at[base + k]],
                             ssems[k], add=True)

    def slab(sl, carry):
        pltpu.sync_copy(srcr.at[wid, sl], src_v)
        pltpu.sync_copy(dstr.at[wid, sl], dst_v)
        group(0, False)

        def body(jj, carry2):
            group(jj * 4, True)
            return carry2

        lax.fori_loop(1, CHS // 4, body, 0)
        pltpu.make_async_copy(r0, agg_s.at[dst_v.at[CHS - 5]], s0_).wait()
        pltpu.async_copy(h_hbm.at[src_v.at[CHS - 1]], r0, g0).wait()
        pltpu.async_copy(r0, agg_s.at[dst_v.at[CHS - 1]], s0_, add=True)
        pltpu.make_async_copy(r0, agg_s.at[dst_v.at[CHS - 1]], s0_).wait()
        pltpu.make_async_copy(r1, agg_s.at[dst_v.at[CHS - 4]], s1_).wait()
        pltpu.make_async_copy(r2, agg_s.at[dst_v.at[CHS - 3]], s2_).wait()
        pltpu.make_async_copy(r3, agg_s.at[dst_v.at[CHS - 2]], s3_).wait()
        return carry

    lax.fori_loop(0, SLABS, slab, 0)
    plsc.subcore_barrier()
    base = c * NP + s * ROWS
    pltpu.sync_copy(agg_s.at[pl.ds(s * ROWS, ROWS)], agg_out.at[pl.ds(base, ROWS)])


_sc_agg = functools.partial(
    pl.kernel,
    out_type=jax.ShapeDtypeStruct((NC * NP, D), jnp.float32),
    mesh=_mesh,
    scratch_types=[
        pltpu.VMEM((CHS, K), jnp.int32),
        pltpu.VMEM((CHS, K), jnp.int32),
        pltpu.VMEM((K, D), jnp.float32),
        pltpu.VMEM((K, D), jnp.float32),
        pltpu.VMEM((K, D), jnp.float32),
        pltpu.VMEM((K, D), jnp.float32),
        pltpu.VMEM_SHARED((NP, D), jnp.float32),
        pltpu.SemaphoreType.DMA,
        pltpu.SemaphoreType.DMA,
        pltpu.SemaphoreType.DMA,
        pltpu.SemaphoreType.DMA,
        pltpu.SemaphoreType.DMA,
        pltpu.SemaphoreType.DMA,
        pltpu.SemaphoreType.DMA,
        pltpu.SemaphoreType.DMA,
    ],
)(_sc_agg_body)


def _leaky(x):
    return jnp.where(x > 0, x, 0.01 * x)


def _row_specs(n):
    return [pl.BlockSpec((BL, D), lambda i: (i, 0)) for _ in range(n)]


_W_SPEC = pl.BlockSpec((D, H), lambda i: (0, 0))
_B_SPEC = pl.BlockSpec((1, H), lambda i: (0, 0))
_A_SPECS = [pl.BlockSpec((1, BL, D), lambda i: (0, i, 0)),
            pl.BlockSpec((1, BL, D), lambda i: (1, i, 0))]
_INV_SPEC = pl.BlockSpec((BL, 8), lambda i: (i, 0))


def _tc_lin_kernel(h_ref, wr_ref, b_ref, hw_ref):
    hw_ref[...] = (jnp.dot(h_ref[...], wr_ref[...],
                           preferred_element_type=jnp.float32) + b_ref[...])


def _tc_lin(h, wr, b2d):
    return pl.pallas_call(
        _tc_lin_kernel,
        grid=(N // BL,),
        in_specs=_row_specs(1) + [_W_SPEC, _B_SPEC],
        out_specs=pl.BlockSpec((BL, H), lambda i: (i, 0)),
        out_shape=jax.ShapeDtypeStruct((N, H), jnp.float32),
    )(h, wr, b2d)


def _tc_inv_kernel(d0_ref, d1_ref, inv_ref):
    deg = d0_ref[0, :, 0:1] + d1_ref[0, :, 0:1]
    inv = 1.0 / jnp.maximum(deg, 1.0)
    inv_ref[...] = jnp.broadcast_to(inv, (BL, 8))


def _tc_inv(degf3):
    return pl.pallas_call(
        _tc_inv_kernel,
        grid=(N // BL,),
        in_specs=_A_SPECS,
        out_specs=_INV_SPEC,
        out_shape=jax.ShapeDtypeStruct((NP, 8), jnp.float32),
    )(degf3, degf3)


def _tc_comb_kernel(hw_ref, a0_ref, a1_ref, inv_ref, wn_ref, h_ref):
    mean = (a0_ref[0] + a1_ref[0]) * inv_ref[:, 0:1]
    h_ref[...] = _leaky(hw_ref[...] + jnp.dot(mean, wn_ref[...],
                                              preferred_element_type=jnp.float32))


def _tc_comb(hw, aggf3, invd, wn):
    return pl.pallas_call(
        _tc_comb_kernel,
        grid=(N // BL,),
        in_specs=_row_specs(1) + _A_SPECS + [_INV_SPEC, _W_SPEC],
        out_specs=pl.BlockSpec((BL, H), lambda i: (i, 0)),
        out_shape=jax.ShapeDtypeStruct((N, H), jnp.float32),
    )(hw, aggf3, aggf3, invd, wn)


def _tc_final_kernel(hw_ref, a0_ref, a1_ref, inv_ref, wn_ref, batch_ref,
                     w1_ref, b1_ref, w2_ref, b2_ref, wc_ref, bc_ref,
                     out_ref, g_acc):
    i = pl.program_id(0)

    @pl.when(i == 0)
    def _():
        g_acc[...] = jnp.zeros_like(g_acc)

    mean = (a0_ref[0] + a1_ref[0]) * inv_ref[:, 0:1]
    h3 = hw_ref[...] + jnp.dot(mean, wn_ref[...],
                               preferred_element_type=jnp.float32)
    seg = jnp.minimum(batch_ref[0, 0, :], B - 1)
    oh = (seg[:, None] ==
          lax.broadcasted_iota(jnp.int32, (1, B), 1)).astype(jnp.float32)
    g_acc[...] += lax.dot_general(oh, h3, (((0,), (0,)), ((), ())),
                                  preferred_element_type=jnp.float32)

    @pl.when(i == pl.num_programs(0) - 1)
    def _():
        g = g_acc[...]
        g = _leaky(jnp.dot(g, w1_ref[...], preferred_element_type=jnp.float32)
                   + b1_ref[...])
        g = _leaky(jnp.dot(g, w2_ref[...], preferred_element_type=jnp.float32)
                   + b2_ref[...])
        out_ref[...] = (jnp.dot(g, wc_ref[...], preferred_element_type=jnp.float32)
                        + bc_ref[...])


def _tc_final(hw, aggf3, invd, wn, batch_r, w1, b1, w2, b2, wc, bc):
    return pl.pallas_call(
        _tc_final_kernel,
        grid=(N // BL,),
        in_specs=_row_specs(1) + _A_SPECS + [
            _INV_SPEC,
            _W_SPEC,
            pl.BlockSpec((1, 1, BL), lambda i: (i, 0, 0)),
            pl.BlockSpec((H, 2 * H), lambda i: (0, 0)),
            pl.BlockSpec((1, 2 * H), lambda i: (0, 0)),
            pl.BlockSpec((2 * H, H), lambda i: (0, 0)),
            _B_SPEC,
            pl.BlockSpec((H, H), lambda i: (0, 0)),
            _B_SPEC,
        ],
        out_specs=pl.BlockSpec((B, H), lambda i: (0, 0)),
        out_shape=jax.ShapeDtypeStruct((B, H), jnp.float32),
        scratch_shapes=[pltpu.VMEM((B, H), jnp.float32)],
    )(hw, aggf3, aggf3, invd, wn, batch_r, w1, b1, w2, b2, wc, bc)


def kernel(x, edge_index, batch, batch_size, Wr, Wn, b,
           lin1_w, lin1_b, lin2_w, lin2_b, linc_w, linc_b):
    src4 = edge_index[0].reshape(NW, SLABS, CHS, K)
    dst4 = edge_index[1].reshape(NW, SLABS, CHS, K)
    dst = edge_index[1].reshape(NW, CH, K)
    z128 = jnp.zeros((NP, D), jnp.float32)
    ones128 = jnp.ones((K, D), jnp.float32)
    batch_r = batch.reshape(N // BL, 1, BL)

    degf3 = _sc_deg(dst, z128, ones128).reshape(2, NP, D)
    hw1 = _tc_lin(x, Wr[0], b[0].reshape(1, H))
    agg13 = _sc_agg(x, src4, dst4, z128).reshape(2, NP, D)
    invd = _tc_inv(degf3)
    h1 = _tc_comb(hw1, agg13, invd, Wn[0])
    hw2 = _tc_lin(h1, Wr[1], b[1].reshape(1, H))
    agg23 = _sc_agg(h1, src4, dst4, z128).reshape(2, NP, D)
    h2 = _tc_comb(hw2, agg23, invd, Wn[1])
    hw3 = _tc_lin(h2, Wr[2], b[2].reshape(1, H))
    agg33 = _sc_agg(h2, src4, dst4, z128).reshape(2, NP, D)
    wc_p = jnp.pad(linc_w, ((0, 0), (0, H - C)))
    bc_p = jnp.pad(linc_b, (0, H - C)).reshape(1, H)
    out_p = _tc_final(hw3, agg33, invd, Wn[2], batch_r,
                      lin1_w, lin1_b.reshape(1, 2 * H),
                      lin2_w, lin2_b.reshape(1, H), wc_p, bc_p)
    return out_p[:, :C]
